# baseline (speedup 1.0000x reference)
import jax
import jax.numpy as jnp
from jax import lax
from jax.experimental import pallas as pl
from jax.experimental.pallas import tpu as pltpu
from jax.experimental.pallas import tpu_sc as plsc

_NUM_EMBED = 1000000
_EMBED_DIM = 32
_BATCH = 16384
_FIELDS = 26

_NC = 2
_NS = 16
_NW = _NC * _NS

_BLK = 128
_NBB = _BATCH // _BLK
_UNITS = _FIELDS * _NBB
_UPW = _UNITS // _NW
_EHI = _EMBED_DIM // 8


def _gather_body(table_hbm, idx_hbm, out_hbm, idxA, idxB, rowsA, rowsB,
                 tA, tB, gsA, gsB, wsA, wsB):
    wid = lax.axis_index("s") * _NC + lax.axis_index("c")
    u0 = wid * _UPW
    iota = lax.iota(jnp.int32, 16)

    def stage_and_fire(g, idxb, rowsb, gsem):
        f = g // _NBB
        bb = g - f * _NBB
        pltpu.sync_copy(idx_hbm.at[f, pl.ds(bb * _BLK, _BLK)], idxb)
        pltpu.async_copy(table_hbm.at[idxb], rowsb, gsem)

    def drain_gather(rowsb, gsem):
        pltpu.make_async_copy(
            table_hbm.at[pl.ds(0, _BLK)], rowsb, gsem).wait()

    def transpose(rowsb, tb):
        for grp in range(8):
            ridx = grp * 16 + iota
            for e in range(_EMBED_DIM):
                v = plsc.load_gather(
                    rowsb, [ridx, jnp.full((16,), e, jnp.int32)])
                tb[e // 8, e % 8, pl.ds(grp * 16, 16)] = v

    def write(g, tb, wsem):
        f = g // _NBB
        bb = g - f * _NBB
        for ehi in range(_EHI):
            pltpu.async_copy(tb.at[ehi], out_hbm.at[f, ehi, bb], wsem)

    def drain_write(tb, wsem):
        pltpu.make_async_copy(
            tb, out_hbm.at[0, pl.ds(0, _EHI), 0], wsem).wait()

    stage_and_fire(u0, idxA, rowsA, gsA)

    @pl.loop(0, _UPW // 2)
    def _(t):
        ua = u0 + 2 * t
        ub = ua + 1
        stage_and_fire(ub, idxB, rowsB, gsB)
        drain_gather(rowsA, gsA)

        @pl.when(t > 0)
        def _():
            drain_write(tA, wsA)

        transpose(rowsA, tA)
        write(ua, tA, wsA)

        @pl.when(t < _UPW // 2 - 1)
        def _():
            stage_and_fire(ub + 1, idxA, rowsA, gsA)

        drain_gather(rowsB, gsB)

        @pl.when(t > 0)
        def _():
            drain_write(tB, wsB)

        transpose(rowsB, tB)
        write(ub, tB, wsB)

    drain_write(tA, wsA)
    drain_write(tB, wsB)


@jax.jit
def _gather(table, idx_t):
    mesh = plsc.VectorSubcoreMesh(core_axis_name="c", subcore_axis_name="s")
    out5 = pl.kernel(
        _gather_body,
        out_type=jax.ShapeDtypeStruct((_FIELDS, _EHI, _NBB, 8, _BLK),
                                      jnp.float32),
        mesh=mesh,
        scratch_types=(
            [pltpu.VMEM((_BLK,), jnp.int32)] * 2
            + [pltpu.VMEM((_BLK, _EMBED_DIM), jnp.float32)] * 2
            + [pltpu.VMEM((_EHI, 8, _BLK), jnp.float32)] * 2
            + [pltpu.SemaphoreType.DMA] * 4
        ),
        compiler_params=pltpu.CompilerParams(use_tc_tiling_on_sc=False,
                                             needs_layout_passes=False),
    )(table, idx_t)
    return out5.transpose(2, 4, 0, 1, 3).reshape(_BATCH, _FIELDS, _EMBED_DIM)


def kernel(x, table):
    return _gather(table, x.T.astype(jnp.int32))


# speedup vs baseline: 1.1248x; 1.1248x over previous
import jax
import jax.numpy as jnp
from jax import lax
from jax.experimental import pallas as pl
from jax.experimental.pallas import tpu as pltpu
from jax.experimental.pallas import tpu_sc as plsc

_NUM_EMBED = 1000000
_EMBED_DIM = 32
_BATCH = 16384
_FIELDS = 26

_NC = 2
_NS = 16
_NW = _NC * _NS

_BLK = 128
_NBB = _BATCH // _BLK
_UNITS = _FIELDS * _NBB
_UPW = _UNITS // _NW
_EHI = _EMBED_DIM // 8


def _gather_body(table_hbm, idx_hbm, out_hbm, idxA, idxB, rowsA, rowsB,
                 tA, tB, gsA, gsB, wsA, wsB):
    wid = lax.axis_index("s") * _NC + lax.axis_index("c")
    u0 = wid * _UPW
    iota = lax.iota(jnp.int32, 16)

    def stage_and_fire(g, idxb, rowsb, gsem):
        f = g // _NBB
        bb = g - f * _NBB
        pltpu.sync_copy(idx_hbm.at[f, pl.ds(bb * _BLK, _BLK)], idxb)
        pltpu.async_copy(table_hbm.at[idxb], rowsb, gsem)

    def drain_gather(rowsb, gsem):
        pltpu.make_async_copy(
            table_hbm.at[pl.ds(0, _BLK)], rowsb, gsem).wait()

    ridx = [grp * 16 + iota for grp in range(8)]

    def transpose(rowsb, tb):
        for e in range(_EMBED_DIM):
            ce = jnp.full((16,), e, jnp.int32)
            vs = [plsc.load_gather(rowsb, [ridx[grp], ce])
                  for grp in range(8)]
            for grp in range(8):
                tb[e // 8, e % 8, pl.ds(grp * 16, 16)] = vs[grp]

    def write(g, tb, wsem):
        f = g // _NBB
        bb = g - f * _NBB
        for ehi in range(_EHI):
            pltpu.async_copy(tb.at[ehi], out_hbm.at[f, ehi, bb], wsem)

    def drain_write(tb, wsem):
        pltpu.make_async_copy(
            tb, out_hbm.at[0, pl.ds(0, _EHI), 0], wsem).wait()

    stage_and_fire(u0, idxA, rowsA, gsA)

    @pl.loop(0, _UPW // 2)
    def _(t):
        ua = u0 + 2 * t
        ub = ua + 1
        stage_and_fire(ub, idxB, rowsB, gsB)
        drain_gather(rowsA, gsA)

        @pl.when(t > 0)
        def _():
            drain_write(tA, wsA)

        transpose(rowsA, tA)
        write(ua, tA, wsA)

        @pl.when(t < _UPW // 2 - 1)
        def _():
            stage_and_fire(ub + 1, idxA, rowsA, gsA)

        drain_gather(rowsB, gsB)

        @pl.when(t > 0)
        def _():
            drain_write(tB, wsB)

        transpose(rowsB, tB)
        write(ub, tB, wsB)

    drain_write(tA, wsA)
    drain_write(tB, wsB)


@jax.jit
def _gather(table, idx_t):
    mesh = plsc.VectorSubcoreMesh(core_axis_name="c", subcore_axis_name="s")
    out5 = pl.kernel(
        _gather_body,
        out_type=jax.ShapeDtypeStruct((_FIELDS, _EHI, _NBB, 8, _BLK),
                                      jnp.float32),
        mesh=mesh,
        scratch_types=(
            [pltpu.VMEM((_BLK,), jnp.int32)] * 2
            + [pltpu.VMEM((_BLK, _EMBED_DIM), jnp.float32)] * 2
            + [pltpu.VMEM((_EHI, 8, _BLK), jnp.float32)] * 2
            + [pltpu.SemaphoreType.DMA] * 4
        ),
        compiler_params=pltpu.CompilerParams(use_tc_tiling_on_sc=False,
                                             needs_layout_passes=False),
    )(table, idx_t)
    return out5.transpose(2, 4, 0, 1, 3).reshape(_BATCH, _FIELDS, _EMBED_DIM)


def kernel(x, table):
    return _gather(table, x.T.astype(jnp.int32))


# speedup vs baseline: 1.1936x; 1.0612x over previous
import jax
import jax.numpy as jnp
from jax import lax
from jax.experimental import pallas as pl
from jax.experimental.pallas import tpu as pltpu
from jax.experimental.pallas import tpu_sc as plsc

_NUM_EMBED = 1000000
_EMBED_DIM = 32
_BATCH = 16384
_FIELDS = 26

_NC = 2
_NS = 16
_NW = _NC * _NS

_BLK = 128
_NBB = _BATCH // _BLK
_UNITS = _FIELDS * _NBB
_UPW = _UNITS // _NW
_EHI = _EMBED_DIM // 8


def _gather_body(table_hbm, idx_hbm, out_hbm, idx_v, rowsA, rowsB,
                 tA, tB, gsA, gsB, wsA, wsB):
    wid = lax.axis_index("s") * _NC + lax.axis_index("c")
    u0 = wid * _UPW
    iota = lax.iota(jnp.int32, 16)
    pltpu.sync_copy(idx_hbm.at[pl.ds(u0 * _BLK, _UPW * _BLK)], idx_v)

    def fire(u, rowsb, gsem):
        pltpu.async_copy(
            table_hbm.at[idx_v.at[pl.ds(u * _BLK, _BLK)]], rowsb, gsem)

    def drain_gather(rowsb, gsem):
        pltpu.make_async_copy(
            table_hbm.at[pl.ds(0, _BLK)], rowsb, gsem).wait()

    ridx = [grp * 16 + iota for grp in range(8)]

    def transpose(rowsb, tb):
        for e in range(_EMBED_DIM):
            ce = jnp.full((16,), e, jnp.int32)
            vs = [plsc.load_gather(rowsb, [ridx[grp], ce])
                  for grp in range(8)]
            for grp in range(8):
                tb[e // 8, e % 8, pl.ds(grp * 16, 16)] = vs[grp]

    def write(g, tb, wsem):
        f = g // _NBB
        bb = g - f * _NBB
        for ehi in range(_EHI):
            pltpu.async_copy(tb.at[ehi], out_hbm.at[f, ehi, bb], wsem)

    def drain_write(tb, wsem):
        pltpu.make_async_copy(
            tb, out_hbm.at[0, pl.ds(0, _EHI), 0], wsem).wait()

    fire(0, rowsA, gsA)

    @pl.loop(0, _UPW // 2)
    def _(t):
        ua = u0 + 2 * t
        ub = ua + 1
        fire(2 * t + 1, rowsB, gsB)
        drain_gather(rowsA, gsA)

        @pl.when(t > 0)
        def _():
            drain_write(tA, wsA)

        transpose(rowsA, tA)
        write(ua, tA, wsA)

        @pl.when(t < _UPW // 2 - 1)
        def _():
            fire(2 * t + 2, rowsA, gsA)

        drain_gather(rowsB, gsB)

        @pl.when(t > 0)
        def _():
            drain_write(tB, wsB)

        transpose(rowsB, tB)
        write(ub, tB, wsB)

    drain_write(tA, wsA)
    drain_write(tB, wsB)


@jax.jit
def _gather(table, idx_t):
    mesh = plsc.VectorSubcoreMesh(core_axis_name="c", subcore_axis_name="s")
    out5 = pl.kernel(
        _gather_body,
        out_type=jax.ShapeDtypeStruct((_FIELDS, _EHI, _NBB, 8, _BLK),
                                      jnp.float32),
        mesh=mesh,
        scratch_types=(
            [pltpu.VMEM((_UPW * _BLK,), jnp.int32)]
            + [pltpu.VMEM((_BLK, _EMBED_DIM), jnp.float32)] * 2
            + [pltpu.VMEM((_EHI, 8, _BLK), jnp.float32)] * 2
            + [pltpu.SemaphoreType.DMA] * 4
        ),
        compiler_params=pltpu.CompilerParams(use_tc_tiling_on_sc=False,
                                             needs_layout_passes=False),
    )(table, idx_t)
    return out5.transpose(2, 4, 0, 1, 3).reshape(_BATCH, _FIELDS, _EMBED_DIM)


def kernel(x, table):
    return _gather(table, x.T.astype(jnp.int32).reshape(-1))


# speedup vs baseline: 1.2009x; 1.0061x over previous
import jax
import jax.numpy as jnp
from jax import lax
from jax.experimental import pallas as pl
from jax.experimental.pallas import tpu as pltpu
from jax.experimental.pallas import tpu_sc as plsc

_NUM_EMBED = 1000000
_EMBED_DIM = 32
_BATCH = 16384
_FIELDS = 26

_NC = 2
_NS = 16
_NW = _NC * _NS

_BLK = 128
_NBB = _BATCH // _BLK
_UNITS = _FIELDS * _NBB
_UPW = _UNITS // _NW
_EHI = _EMBED_DIM // 8


def _gather_body(table_hbm, idx_hbm, out_hbm, idx_v, rowsA, rowsB,
                 tA, tB, gsA, gsB, wsA, wsB):
    wid = lax.axis_index("s") * _NC + lax.axis_index("c")
    u0 = wid * _UPW
    iota = lax.iota(jnp.int32, 16)
    pltpu.sync_copy(idx_hbm.at[pl.ds(u0 * _BLK, _UPW * _BLK)], idx_v)

    def fire(u, rowsb, gsem):
        pltpu.async_copy(
            table_hbm.at[idx_v.at[pl.ds(u * _BLK, _BLK)]], rowsb, gsem)

    def drain_gather(rowsb, gsem):
        pltpu.make_async_copy(
            table_hbm.at[pl.ds(0, _BLK)], rowsb, gsem).wait()

    ridx = [grp * 16 + iota for grp in range(8)]

    def transpose(rowsb, tb):
        for e0 in range(0, _EMBED_DIM, 2):
            ces = [jnp.full((16,), e0 + d, jnp.int32) for d in range(2)]
            vs = [plsc.load_gather(rowsb, [ridx[grp], ces[d]])
                  for d in range(2) for grp in range(8)]
            for d in range(2):
                for grp in range(8):
                    tb[e0 + d, pl.ds(grp * 16, 16)] = vs[d * 8 + grp]

    def write(g, tb, wsem):
        f = g // _NBB
        bb = g - f * _NBB
        for ehi in range(_EHI):
            pltpu.async_copy(tb.at[pl.ds(ehi * 8, 8)],
                             out_hbm.at[f, ehi, bb], wsem)

    def drain_write(tb, wsem):
        pltpu.make_async_copy(
            tb, out_hbm.at[0, pl.ds(0, _EHI), 0], wsem).wait()

    fire(0, rowsA, gsA)

    @pl.loop(0, _UPW // 2)
    def _(t):
        ua = u0 + 2 * t
        ub = ua + 1
        fire(2 * t + 1, rowsB, gsB)
        drain_gather(rowsA, gsA)

        @pl.when(t > 0)
        def _():
            drain_write(tA, wsA)

        transpose(rowsA, tA)
        write(ua, tA, wsA)

        @pl.when(t < _UPW // 2 - 1)
        def _():
            fire(2 * t + 2, rowsA, gsA)

        drain_gather(rowsB, gsB)

        @pl.when(t > 0)
        def _():
            drain_write(tB, wsB)

        transpose(rowsB, tB)
        write(ub, tB, wsB)

    drain_write(tA, wsA)
    drain_write(tB, wsB)


@jax.jit
def _gather(table, idx_t):
    mesh = plsc.VectorSubcoreMesh(core_axis_name="c", subcore_axis_name="s")
    out5 = pl.kernel(
        _gather_body,
        out_type=jax.ShapeDtypeStruct((_FIELDS, _EHI, _NBB, 8, _BLK),
                                      jnp.float32),
        mesh=mesh,
        scratch_types=(
            [pltpu.VMEM((_UPW * _BLK,), jnp.int32)]
            + [pltpu.VMEM((_BLK, _EMBED_DIM), jnp.float32)] * 2
            + [pltpu.VMEM((_EMBED_DIM, _BLK), jnp.float32)] * 2
            + [pltpu.SemaphoreType.DMA] * 4
        ),
        compiler_params=pltpu.CompilerParams(use_tc_tiling_on_sc=False,
                                             needs_layout_passes=False),
    )(table, idx_t)
    return out5.transpose(2, 4, 0, 1, 3).reshape(_BATCH, _FIELDS, _EMBED_DIM)


def kernel(x, table):
    return _gather(table, x.T.astype(jnp.int32).reshape(-1))


# speedup vs baseline: 1.2177x; 1.0140x over previous
import jax
import jax.numpy as jnp
from jax import lax
from jax.experimental import pallas as pl
from jax.experimental.pallas import tpu as pltpu
from jax.experimental.pallas import tpu_sc as plsc

_NUM_EMBED = 1000000
_EMBED_DIM = 32
_BATCH = 16384
_FIELDS = 26

_NC = 2
_NS = 16
_NW = _NC * _NS

_BLK = 128
_NBB = _BATCH // _BLK
_UNITS = _FIELDS * _NBB
_UPW = _UNITS // _NW
_EHI = _EMBED_DIM // 8


def _gather_body(table_hbm, idx_hbm, out_hbm, idx_v, rowsA, rowsB,
                 tA, tB, gsA, gsB, wsA, wsB):
    wid = lax.axis_index("s") * _NC + lax.axis_index("c")
    u0 = wid * _UPW
    iota = lax.iota(jnp.int32, 16)
    pltpu.sync_copy(idx_hbm.at[pl.ds(u0 * _BLK, _UPW * _BLK)], idx_v)

    def fire(u, rowsb, gsem):
        pltpu.async_copy(
            table_hbm.at[idx_v.at[pl.ds(u * _BLK, _BLK)]], rowsb, gsem)

    def drain_gather(rowsb, gsem):
        pltpu.make_async_copy(
            table_hbm.at[pl.ds(0, _BLK)], rowsb, gsem).wait()

    ridx = [grp * 16 + iota for grp in range(8)]

    def transpose(rowsb, tb):
        for e0 in range(0, _EMBED_DIM, 2):
            ces = [jnp.full((16,), e0 + d, jnp.int32) for d in range(2)]
            vs = [plsc.load_gather(rowsb, [ridx[grp], ces[d]])
                  for d in range(2) for grp in range(8)]
            for d in range(2):
                for grp in range(8):
                    tb[e0 + d, pl.ds(grp * 16, 16)] = vs[d * 8 + grp]

    def write(g, tb, wsem):
        f = g // _NBB
        bb = g - f * _NBB
        for ehi in range(_EHI):
            pltpu.async_copy(tb.at[pl.ds(ehi * 8, 8)],
                             out_hbm.at[f, ehi, bb], wsem)

    def drain_write(tb, wsem):
        pltpu.make_async_copy(
            tb, out_hbm.at[0, pl.ds(0, _EHI), 0], wsem).wait()

    fire(0, rowsA, gsA)

    @pl.loop(0, _UPW // 2)
    def _(t):
        ua = u0 + 2 * t
        ub = ua + 1
        fire(2 * t + 1, rowsB, gsB)
        drain_gather(rowsA, gsA)

        @pl.when(t > 0)
        def _():
            drain_write(tA, wsA)

        transpose(rowsA, tA)
        write(ua, tA, wsA)

        @pl.when(t < _UPW // 2 - 1)
        def _():
            fire(2 * t + 2, rowsA, gsA)

        drain_gather(rowsB, gsB)

        @pl.when(t > 0)
        def _():
            drain_write(tB, wsB)

        transpose(rowsB, tB)
        write(ub, tB, wsB)

    drain_write(tA, wsA)
    drain_write(tB, wsB)


@jax.jit
def _gather(table, idx_t):
    mesh = plsc.VectorSubcoreMesh(core_axis_name="c", subcore_axis_name="s")
    out5 = pl.kernel(
        _gather_body,
        out_type=jax.ShapeDtypeStruct((_FIELDS, _EHI, _NBB, 8, _BLK),
                                      jnp.float32),
        mesh=mesh,
        scratch_types=(
            [pltpu.VMEM((_UPW * _BLK,), jnp.int32)]
            + [pltpu.VMEM((_BLK, 128), jnp.float32)] * 2
            + [pltpu.VMEM((_EMBED_DIM, _BLK), jnp.float32)] * 2
            + [pltpu.SemaphoreType.DMA] * 4
        ),
        compiler_params=pltpu.CompilerParams(use_tc_tiling_on_sc=False,
                                             needs_layout_passes=False),
    )(table, idx_t)
    return out5.transpose(2, 4, 0, 1, 3).reshape(_BATCH, _FIELDS, _EMBED_DIM)


def kernel(x, table):
    tpad = jnp.pad(table, ((0, 0), (0, 96)))
    return _gather(tpad, x.T.astype(jnp.int32).reshape(-1))


# speedup vs baseline: 1.2185x; 1.0006x over previous
import jax
import jax.numpy as jnp
from jax import lax
from jax.experimental import pallas as pl
from jax.experimental.pallas import tpu as pltpu
from jax.experimental.pallas import tpu_sc as plsc

_NUM_EMBED = 1000000
_EMBED_DIM = 32
_BATCH = 16384
_FIELDS = 26
_PADW = 128

_NC = 2
_NS = 16
_NW = _NC * _NS

_BLK = 128
_NBB = _BATCH // _BLK
_UNITS = _FIELDS * _NBB
_UPW = _UNITS // _NW
_EHI = _EMBED_DIM // 8
_NRB = 4
_QUADS = _UPW // _NRB


def _gather_body(table_hbm, idx_hbm, out_hbm, idx_v, *rest):
    rows = rest[:_NRB]
    tb = rest[_NRB:_NRB + 2]
    gs = rest[_NRB + 2:2 * _NRB + 2]
    ws = rest[2 * _NRB + 2:2 * _NRB + 4]

    wid = lax.axis_index("s") * _NC + lax.axis_index("c")
    u0 = wid * _UPW
    iota = lax.iota(jnp.int32, 16)
    pltpu.sync_copy(idx_hbm.at[pl.ds(u0 * _BLK, _UPW * _BLK)], idx_v)

    def fire(u, j):
        pltpu.async_copy(
            table_hbm.at[idx_v.at[pl.ds(u * _BLK, _BLK)]], rows[j], gs[j])

    def drain_gather(j):
        pltpu.make_async_copy(
            table_hbm.at[pl.ds(0, _BLK)], rows[j], gs[j]).wait()

    ridx = [grp * 16 + iota for grp in range(8)]

    def transpose(j, p):
        for e0 in range(0, _EMBED_DIM, 2):
            ces = [jnp.full((16,), e0 + d, jnp.int32) for d in range(2)]
            vs = [plsc.load_gather(rows[j], [ridx[grp], ces[d]])
                  for d in range(2) for grp in range(8)]
            for d in range(2):
                for grp in range(8):
                    tb[p][e0 + d, pl.ds(grp * 16, 16)] = vs[d * 8 + grp]

    def write(g, p):
        f = g // _NBB
        bb = g - f * _NBB
        for ehi in range(_EHI):
            pltpu.async_copy(tb[p].at[pl.ds(ehi * 8, 8)],
                             out_hbm.at[f, ehi, bb], ws[p])

    def drain_write(p):
        pltpu.make_async_copy(
            tb[p], out_hbm.at[0, pl.ds(0, _EHI), 0], ws[p]).wait()

    for j in range(_NRB - 1):
        fire(j, j)

    @pl.loop(0, _QUADS - 1)
    def _(t):
        for j in range(_NRB):
            u = _NRB * t + j
            fire(u + _NRB - 1, (j + _NRB - 1) % _NRB)
            drain_gather(j)

            @pl.when(u > 1)
            def _():
                drain_write(j % 2)

            transpose(j, j % 2)
            write(u0 + u, j % 2)

    base = _NRB * (_QUADS - 1)
    fire(base + _NRB - 1, _NRB - 1)
    for j in range(_NRB):
        u = base + j
        drain_gather(j)
        drain_write(j % 2)
        transpose(j, j % 2)
        write(u0 + u, j % 2)
    drain_write(0)
    drain_write(1)


@jax.jit
def _gather(table, idx_t):
    mesh = plsc.VectorSubcoreMesh(core_axis_name="c", subcore_axis_name="s")
    out5 = pl.kernel(
        _gather_body,
        out_type=jax.ShapeDtypeStruct((_FIELDS, _EHI, _NBB, 8, _BLK),
                                      jnp.float32),
        mesh=mesh,
        scratch_types=(
            [pltpu.VMEM((_UPW * _BLK,), jnp.int32)]
            + [pltpu.VMEM((_BLK, _PADW), jnp.float32)] * _NRB
            + [pltpu.VMEM((_EMBED_DIM, _BLK), jnp.float32)] * 2
            + [pltpu.SemaphoreType.DMA] * (_NRB + 2)
        ),
        compiler_params=pltpu.CompilerParams(use_tc_tiling_on_sc=False,
                                             needs_layout_passes=False),
    )(table, idx_t)
    return out5.transpose(2, 4, 0, 1, 3).reshape(_BATCH, _FIELDS, _EMBED_DIM)


def kernel(x, table):
    tpad = jnp.pad(table, ((0, 0), (0, _PADW - _EMBED_DIM)))
    return _gather(tpad, x.T.astype(jnp.int32).reshape(-1))


# speedup vs baseline: 1.2219x; 1.0029x over previous
import jax
import jax.numpy as jnp
from jax import lax
from jax.experimental import pallas as pl
from jax.experimental.pallas import tpu as pltpu
from jax.experimental.pallas import tpu_sc as plsc

_NUM_EMBED = 1000000
_EMBED_DIM = 32
_BATCH = 16384
_FIELDS = 26
_PADW = 128

_NC = 2
_NS = 16
_NW = _NC * _NS

_BLK = 128
_NBB = _BATCH // _BLK
_UNITS = _FIELDS * _NBB
_UPW = _UNITS // _NW
_EHI = _EMBED_DIM // 8
_NRB = 4
_QUADS = _UPW // _NRB


def _gather_body(table_hbm, idx_hbm, out_hbm, idx_v, *rest):
    rows = rest[:_NRB]
    tb = rest[_NRB:_NRB + 2]
    gs = rest[_NRB + 2:2 * _NRB + 2]
    ws = rest[2 * _NRB + 2:2 * _NRB + 4]

    wid = lax.axis_index("s") * _NC + lax.axis_index("c")
    u0 = wid * _UPW
    iota = lax.iota(jnp.int32, 16)
    pltpu.sync_copy(idx_hbm.at[pl.ds(u0 * _BLK, _UPW * _BLK)], idx_v)

    def fire(u, j):
        pltpu.async_copy(
            table_hbm.at[idx_v.at[pl.ds(u * _BLK, _BLK)]], rows[j], gs[j])

    def drain_gather(j):
        pltpu.make_async_copy(
            table_hbm.at[pl.ds(0, _BLK)], rows[j], gs[j]).wait()

    ridx = [grp * 16 + iota for grp in range(8)]

    def transpose(j, p):
        for e0 in range(0, _EMBED_DIM, 2):
            ces = [jnp.full((16,), e0 + d, jnp.int32) for d in range(2)]
            vs = [plsc.load_gather(rows[j], [ridx[grp], ces[d]])
                  for d in range(2) for grp in range(8)]
            for d in range(2):
                for grp in range(8):
                    tb[p][e0 + d, pl.ds(grp * 16, 16)] = vs[d * 8 + grp]

    def write(g, p):
        f = g // _NBB
        bb = g - f * _NBB
        for ehi in range(_EHI):
            pltpu.async_copy(tb[p].at[pl.ds(ehi * 8, 8)],
                             out_hbm.at[f, ehi, bb], ws[p])

    def drain_write(p):
        pltpu.make_async_copy(
            tb[p], out_hbm.at[0, pl.ds(0, _EHI), 0], ws[p]).wait()

    for j in range(_NRB - 1):
        fire(j, j)

    @pl.loop(0, _QUADS - 1)
    def _(t):
        for j in range(_NRB):
            u = _NRB * t + j
            fire(u + _NRB - 1, (j + _NRB - 1) % _NRB)
            drain_gather(j)

            @pl.when(u > 1)
            def _():
                drain_write(j % 2)

            transpose(j, j % 2)
            write(u0 + u, j % 2)

    base = _NRB * (_QUADS - 1)
    fire(base + _NRB - 1, _NRB - 1)
    for j in range(_NRB):
        u = base + j
        drain_gather(j)
        drain_write(j % 2)
        transpose(j, j % 2)
        write(u0 + u, j % 2)
    drain_write(0)
    drain_write(1)


@jax.jit
def _gather(table, idx_t):
    mesh = plsc.VectorSubcoreMesh(core_axis_name="c", subcore_axis_name="s")
    out5 = pl.kernel(
        _gather_body,
        out_type=jax.ShapeDtypeStruct((_FIELDS, _EHI, _NBB, 8, _BLK),
                                      jnp.float32),
        mesh=mesh,
        scratch_types=(
            [pltpu.VMEM((_UPW * _BLK,), jnp.int32)]
            + [pltpu.VMEM((_BLK, _EMBED_DIM), jnp.float32)] * _NRB
            + [pltpu.VMEM((_EMBED_DIM, _BLK), jnp.float32)] * 2
            + [pltpu.SemaphoreType.DMA] * (_NRB + 2)
        ),
        compiler_params=pltpu.CompilerParams(use_tc_tiling_on_sc=False,
                                             needs_layout_passes=False),
    )(table, idx_t)
    return out5.transpose(2, 4, 0, 1, 3).reshape(_BATCH, _FIELDS, _EMBED_DIM)


def kernel(x, table):
    tpad = jnp.pad(table, ((0, 0), (0, _PADW - _EMBED_DIM)))
    t4 = tpad.reshape(_NUM_EMBED * (_PADW // _EMBED_DIM), _EMBED_DIM)
    idx4 = x.T.astype(jnp.int32).reshape(-1) * (_PADW // _EMBED_DIM)
    return _gather(t4, idx4)


# speedup vs baseline: 1.2535x; 1.0258x over previous
import jax
import jax.numpy as jnp
from jax import lax
from jax.experimental import pallas as pl
from jax.experimental.pallas import tpu as pltpu
from jax.experimental.pallas import tpu_sc as plsc

_NUM_EMBED = 1000000
_EMBED_DIM = 32
_BATCH = 16384
_FIELDS = 26
_PADW = 128

_NC = 2
_NS = 16
_NW = _NC * _NS

_BLK = 128
_NBB = _BATCH // _BLK
_UNITS = _FIELDS * _NBB
_UPW = _UNITS // _NW
_EHI = _EMBED_DIM // 8
_NRB = 4
_QUADS = _UPW // _NRB


def _gather_body(table_hbm, idx_hbm, out_hbm, idx_v, *rest):
    rows = rest[:_NRB]
    tb = rest[_NRB:_NRB + 2]
    gs = rest[_NRB + 2:2 * _NRB + 2]
    ws = rest[2 * _NRB + 2:2 * _NRB + 4]

    wid = lax.axis_index("s") * _NC + lax.axis_index("c")
    u0 = wid * _UPW
    iota = lax.iota(jnp.int32, 16)
    pltpu.sync_copy(idx_hbm.at[pl.ds(u0 * _BLK, _UPW * _BLK)], idx_v)

    def fire(u, j):
        pltpu.async_copy(
            table_hbm.at[idx_v.at[pl.ds(u * _BLK, _BLK)]], rows[j], gs[j])

    def drain_gather(j):
        pltpu.make_async_copy(
            table_hbm.at[pl.ds(0, _BLK)], rows[j], gs[j]).wait()

    ridx = [grp * 16 + iota for grp in range(8)]

    def transpose(j, p):
        @plsc.parallel_loop(0, _EMBED_DIM, unroll=4)
        def _(e):
            ce = jnp.full((16,), e, jnp.int32)
            vs = [plsc.load_gather(rows[j], [ridx[grp], ce])
                  for grp in range(8)]
            for grp in range(8):
                tb[p][e, pl.ds(grp * 16, 16)] = vs[grp]

    def write(g, p):
        f = g // _NBB
        bb = g - f * _NBB
        for ehi in range(_EHI):
            pltpu.async_copy(tb[p].at[pl.ds(ehi * 8, 8)],
                             out_hbm.at[f, ehi, bb], ws[p])

    def drain_write(p):
        pltpu.make_async_copy(
            tb[p], out_hbm.at[0, pl.ds(0, _EHI), 0], ws[p]).wait()

    for j in range(_NRB - 1):
        fire(j, j)

    @pl.loop(0, _QUADS - 1)
    def _(t):
        for j in range(_NRB):
            u = _NRB * t + j
            fire(u + _NRB - 1, (j + _NRB - 1) % _NRB)
            drain_gather(j)

            @pl.when(u > 1)
            def _():
                drain_write(j % 2)

            transpose(j, j % 2)
            write(u0 + u, j % 2)

    base = _NRB * (_QUADS - 1)
    fire(base + _NRB - 1, _NRB - 1)
    for j in range(_NRB):
        u = base + j
        drain_gather(j)
        drain_write(j % 2)
        transpose(j, j % 2)
        write(u0 + u, j % 2)
    drain_write(0)
    drain_write(1)


@jax.jit
def _gather(table, idx_t):
    mesh = plsc.VectorSubcoreMesh(core_axis_name="c", subcore_axis_name="s")
    out5 = pl.kernel(
        _gather_body,
        out_type=jax.ShapeDtypeStruct((_FIELDS, _EHI, _NBB, 8, _BLK),
                                      jnp.float32),
        mesh=mesh,
        scratch_types=(
            [pltpu.VMEM((_UPW * _BLK,), jnp.int32)]
            + [pltpu.VMEM((_BLK, _EMBED_DIM), jnp.float32)] * _NRB
            + [pltpu.VMEM((_EMBED_DIM, _BLK), jnp.float32)] * 2
            + [pltpu.SemaphoreType.DMA] * (_NRB + 2)
        ),
        compiler_params=pltpu.CompilerParams(use_tc_tiling_on_sc=False,
                                             needs_layout_passes=False),
    )(table, idx_t)
    return out5.transpose(2, 4, 0, 1, 3).reshape(_BATCH, _FIELDS, _EMBED_DIM)


def kernel(x, table):
    tpad = jnp.pad(table, ((0, 0), (0, _PADW - _EMBED_DIM)))
    t4 = tpad.reshape(_NUM_EMBED * (_PADW // _EMBED_DIM), _EMBED_DIM)
    idx4 = x.T.astype(jnp.int32).reshape(-1) * (_PADW // _EMBED_DIM)
    return _gather(t4, idx4)
